# Initial kernel scaffold; baseline (speedup 1.0000x reference)
import jax
import jax.numpy as jnp
from jax.experimental import pallas as pl


def kernel(species, embedding, idx0, idx1, idx2, idx3, W1, b1, W2, b2, W3, b3):
    raise NotImplementedError("write your pallas kernel here")


# speedup vs baseline: 2.8505x; 2.8505x over previous
import functools

import jax
import jax.numpy as jnp
from jax import lax
from jax.experimental import pallas as pl
from jax.experimental.pallas import tpu as pltpu
from jax.experimental.pallas import tpu_sc as plsc

N = 32768
S = 4
D_IN = 256
D_H = 512
D_OUT = 256

NC = 2
NS = 16
NW = NC * NS
ROWS_PER_W = N // NW
CH = 128
NCH = ROWS_PER_W // CH

_mesh = plsc.VectorSubcoreMesh(core_axis_name="c", subcore_axis_name="s")


def _worker_base():
    wid = lax.axis_index("s") * NC + lax.axis_index("c")
    return wid, wid * ROWS_PER_W


@functools.partial(
    pl.kernel,
    out_type=jax.ShapeDtypeStruct((N, D_IN), jnp.float32),
    mesh=_mesh,
    scratch_types=[
        pltpu.VMEM((NCH, CH), jnp.int32),
        pltpu.VMEM((CH, D_IN), jnp.float32),
        pltpu.VMEM((CH, D_IN), jnp.float32),
        pltpu.SemaphoreType.DMA,
        pltpu.SemaphoreType.DMA,
    ],
)
def _sc_gather(emb_hbm, idx_hbm, xg_hbm, idx_v, buf0, buf1, sem0, sem1):
    wid, base = _worker_base()
    pltpu.sync_copy(idx_hbm.at[wid], idx_v)
    bufs = (buf0, buf1)
    sems = (sem0, sem1)
    cps = [
        pltpu.async_copy(emb_hbm.at[idx_v.at[0]], buf0, sem0),
        pltpu.async_copy(emb_hbm.at[idx_v.at[1]], buf1, sem1),
    ]
    for c in range(NCH):
        b = c % 2
        cps[b].wait()
        pltpu.sync_copy(bufs[b], xg_hbm.at[pl.ds(base + c * CH, CH)])
        if c + 2 < NCH:
            cps[b] = pltpu.async_copy(emb_hbm.at[idx_v.at[c + 2]], bufs[b], sems[b])


@functools.partial(
    pl.kernel,
    out_type=jax.ShapeDtypeStruct((N, D_OUT), jnp.float32),
    mesh=_mesh,
    scratch_types=[
        pltpu.VMEM((NCH, CH), jnp.int32),
        pltpu.VMEM((CH, D_OUT), jnp.float32),
        pltpu.VMEM((CH, D_OUT), jnp.float32),
        pltpu.SemaphoreType.DMA,
        pltpu.SemaphoreType.DMA,
    ],
)
def _sc_scatter(o_hbm, idx_hbm, out_hbm, idx_v, buf0, buf1, sem0, sem1):
    wid, base = _worker_base()
    pltpu.sync_copy(idx_hbm.at[wid], idx_v)
    bufs = (buf0, buf1)
    sems = (sem0, sem1)
    pltpu.sync_copy(o_hbm.at[pl.ds(base, CH)], buf0)
    prev = None
    for c in range(NCH):
        b = c % 2
        if c + 1 < NCH:
            pltpu.sync_copy(o_hbm.at[pl.ds(base + (c + 1) * CH, CH)], bufs[1 - b])
        if prev is not None:
            prev.wait()
        prev = pltpu.async_copy(bufs[b], out_hbm.at[idx_v.at[c]], sems[b])
    prev.wait()


BM = 512
MB = (N // S) // BM


def _mlp_body(x_ref, w1_ref, b1_ref, w2_ref, b2_ref, w3_ref, b3_ref, o_ref):
    x = x_ref[...]
    h = jnp.dot(x, w1_ref[0], preferred_element_type=jnp.float32) + b1_ref[0]
    h = h * jax.nn.sigmoid(h)
    h = jnp.dot(h, w2_ref[0], preferred_element_type=jnp.float32) + b2_ref[0]
    h = h * jax.nn.sigmoid(h)
    o_ref[...] = jnp.dot(h, w3_ref[0], preferred_element_type=jnp.float32) + b3_ref[0]


_mlp = pl.pallas_call(
    _mlp_body,
    grid=(S, MB),
    in_specs=[
        pl.BlockSpec((BM, D_IN), lambda s, m: (s * MB + m, 0)),
        pl.BlockSpec((1, D_IN, D_H), lambda s, m: (s, 0, 0)),
        pl.BlockSpec((1, 1, D_H), lambda s, m: (s, 0, 0)),
        pl.BlockSpec((1, D_H, D_H), lambda s, m: (s, 0, 0)),
        pl.BlockSpec((1, 1, D_H), lambda s, m: (s, 0, 0)),
        pl.BlockSpec((1, D_H, D_OUT), lambda s, m: (s, 0, 0)),
        pl.BlockSpec((1, 1, D_OUT), lambda s, m: (s, 0, 0)),
    ],
    out_specs=pl.BlockSpec((BM, D_OUT), lambda s, m: (s * MB + m, 0)),
    out_shape=jax.ShapeDtypeStruct((N, D_OUT), jnp.float32),
    compiler_params=pltpu.CompilerParams(
        dimension_semantics=("arbitrary", "arbitrary"),
    ),
)


@jax.jit
def kernel(species, embedding, idx0, idx1, idx2, idx3, W1, b1, W2, b2, W3, b3):
    del species
    idx_all = jnp.concatenate([idx0, idx1, idx2, idx3]).astype(jnp.int32)
    idx3d = idx_all.reshape(NW, NCH, CH)
    xg = _sc_gather(embedding, idx3d)
    o = _mlp(
        xg,
        W1,
        b1.reshape(S, 1, D_H),
        W2,
        b2.reshape(S, 1, D_H),
        W3,
        b3.reshape(S, 1, D_OUT),
    )
    return _sc_scatter(o, idx3d)
